# baseline (speedup 1.0000x reference)
import jax
import jax.numpy as jnp
from jax import lax
from jax.experimental import pallas as pl
from jax.experimental.pallas import tpu as pltpu
from jax.experimental.pallas import tpu_sc as plsc

N = 10000
E = 320000
D = 128
H = 64
C = 128
NC = 2
NS = 16
NW = NC * NS
PAD_ROWS = 240
NP = N + PAD_ROWS
W = 128
WINS = 80
EPW = W * WINS
EP = NW * EPW
NROWS_TILE = NP // NS
BE = 2048
BN = 1000
NCHUNK = 5
WINS_CK = WINS // NCHUNK
EPW_CK = W * WINS_CK
EP_CK = EP // NCHUNK


def _full_spec(shape):
    return pl.BlockSpec(shape, lambda i: tuple(0 for _ in shape))


def _preproj_body(h_ref, w_ref, ab_ref):
    ab_ref[...] = jnp.dot(h_ref[...], w_ref[...],
                          preferred_element_type=jnp.float32)


def _preproj(h, w):
    return pl.pallas_call(
        _preproj_body,
        grid=(N // BN,),
        in_specs=[
            pl.BlockSpec((BN, D), lambda i: (i, 0)),
            _full_spec((D, C)),
        ],
        out_specs=pl.BlockSpec((BN, C), lambda i: (i, 0)),
        out_shape=jax.ShapeDtypeStruct((N, C), jnp.float32),
    )(h, w)


def _sc_gather_body(ab_hbm, xt0_hbm, xt1_hbm, xt2_hbm, rowp_hbm, colp_hbm,
                    pbc_hbm, g_hbm,
                    idxr_v, idxc_v, xp0, xp1, xp2, pbc_v,
                    abrow0, abrow1, abcol0, abcol1,
                    sema0, sema1, semb0, semb1):
    cid = lax.axis_index("c")
    sid = lax.axis_index("s")
    wid = sid * NC + cid
    wbase = wid * WINS_CK
    pltpu.sync_copy(rowp_hbm.at[pl.ds(wbase, WINS_CK)], idxr_v)
    pltpu.sync_copy(colp_hbm.at[pl.ds(wbase, WINS_CK)], idxc_v)
    pltpu.sync_copy(xt0_hbm, xp0)
    pltpu.sync_copy(xt1_hbm, xp1)
    pltpu.sync_copy(xt2_hbm, xp2)
    pltpu.sync_copy(pbc_hbm, pbc_v)
    box = tuple(pbc_v[pl.ds(p * 16, 16)] for p in range(3))
    ibox = tuple(pbc_v[pl.ds(48 + p * 16, 16)] for p in range(3))
    magic = jnp.full((16,), 12582912.0, jnp.float32)

    abrow = (abrow0, abrow1)
    abcol = (abcol0, abcol1)
    sema = (sema0, sema1)
    semb = (semb0, semb1)

    def fire(j, b):
        pltpu.async_copy(ab_hbm.at[idxr_v.at[j]], abrow[b], sema[b])
        pltpu.async_copy(ab_hbm.at[idxc_v.at[j]], abcol[b], semb[b])

    def drain(b):
        pltpu.make_async_copy(ab_hbm.at[idxr_v.at[0]], abrow[b],
                              sema[b]).wait()
        pltpu.make_async_copy(ab_hbm.at[idxc_v.at[0]], abcol[b],
                              semb[b]).wait()

    fire(0, 0)

    def process(j, b):
        drain(b)

        def addrows(r4, c2):
            for dr in range(4):
                r = r4 * 4 + dr
                for q in range(H // 16):
                    plsc.addupdate(abrow[b].at[r, pl.ds(q * 16, 16)],
                                   abcol[b][r, pl.ds(H + q * 16, 16)])
            return c2

        lax.fori_loop(0, W // 4, addrows, 0)

        for g in range(W // 16):
            s = pl.ds(g * 16, 16)
            ev = jnp.arange(g * 16, g * 16 + 16, dtype=jnp.int32)
            rv = idxr_v[j, s]
            cv = idxc_v[j, s]
            d = []
            for p, plane in enumerate((xp0, xp1, xp2)):
                dp = plsc.load_gather(plane, [rv]) \
                    - plsc.load_gather(plane, [cv])
                k = (dp * ibox[p] + magic) - magic
                dp = dp - box[p] * k
                d.append(dp)
                pv = jnp.full((16,), H + p, dtype=jnp.int32)
                plsc.store_scatter(abrow[b], [ev, pv], dp)
            r2 = d[0] * d[0] + d[1] * d[1] + d[2] * d[2]
            pv = jnp.full((16,), H + 3, dtype=jnp.int32)
            plsc.store_scatter(abrow[b], [ev, pv], r2)
        ebase = wid * EPW_CK + j * W
        pltpu.sync_copy(abrow[b], g_hbm.at[pl.ds(ebase, W)])

    def superwin(js, carry):
        for bb in range(2):
            j = js * 2 + bb
            jn = jnp.minimum(j + 1, WINS_CK - 1)
            fire(jn, (bb + 1) % 2)
            process(j, bb)
        return carry

    lax.fori_loop(0, WINS_CK // 2, superwin, 0)
    drain(0)


def _sc_gather(abp, xtp, rowp, colp, pbc):
    mesh = plsc.VectorSubcoreMesh(core_axis_name="c", subcore_axis_name="s")
    fn = pl.kernel(
        _sc_gather_body,
        out_type=jax.ShapeDtypeStruct((EP_CK, C), jnp.float32),
        mesh=mesh,
        scratch_types=[
            pltpu.VMEM((WINS_CK, W), jnp.int32),
            pltpu.VMEM((WINS_CK, W), jnp.int32),
            pltpu.VMEM((NP,), jnp.float32),
            pltpu.VMEM((NP,), jnp.float32),
            pltpu.VMEM((NP,), jnp.float32),
            pltpu.VMEM((96,), jnp.float32),
            pltpu.VMEM((W, C), jnp.float32),
            pltpu.VMEM((W, C), jnp.float32),
            pltpu.VMEM((W, C), jnp.float32),
            pltpu.VMEM((W, C), jnp.float32),
            pltpu.SemaphoreType.DMA,
            pltpu.SemaphoreType.DMA,
            pltpu.SemaphoreType.DMA,
            pltpu.SemaphoreType.DMA,
        ],
        compiler_params=pltpu.CompilerParams(needs_layout_passes=False),
    )
    return fn(abp, xtp[0], xtp[1], xtp[2], rowp, colp, pbc)


def _edge_body(g_ref, we1r_ref, be1_ref, w2t_ref, be2_ref,
               wc1t_ref, bc1_ref, wc2t_ref, et_ref):
    bf16 = jnp.bfloat16

    def silu(v):
        return v * (0.5 * jnp.tanh(0.5 * v) + 0.5)

    gblk = g_ref[...]
    d = gblk[:, H:H + 3]
    r2c = gblk[:, H + 3:H + 4]
    r2f = jnp.dot(r2c, jnp.ones((1, H), jnp.float32),
                  preferred_element_type=jnp.float32)
    normf = jnp.sqrt(r2f + 1e-8)
    pre = gblk[:, 0:H] + normf * we1r_ref[...] + be1_ref[...]
    t1 = silu(pre)
    t2 = jnp.dot(t1.astype(bf16), w2t_ref[...],
                 preferred_element_type=jnp.float32) + be2_ref[...]
    ef = silu(t2)
    s1 = jnp.dot(ef.astype(bf16), wc1t_ref[...],
                 preferred_element_type=jnp.float32) + bc1_ref[...]
    s1 = silu(s1)
    scale = jnp.dot(s1, wc2t_ref[...], preferred_element_type=jnp.float32)
    m = scale / (normf[:, 0:1] + 1.0)
    tr = d * m
    et_ref[...] = jnp.concatenate(
        [ef, tr, jnp.zeros((ef.shape[0], C - H - 3), jnp.float32)], axis=1)


def _edge_mlp(g, we1r, be1r, w2t, be2r, wc1t, bc1r, wc2t):
    return pl.pallas_call(
        _edge_body,
        grid=(EP_CK // BE,),
        in_specs=[
            pl.BlockSpec((BE, C), lambda i: (i, 0)),
            _full_spec((1, H)),
            _full_spec((1, H)),
            _full_spec((H, H)),
            _full_spec((1, H)),
            _full_spec((H, H)),
            _full_spec((1, H)),
            _full_spec((H, 1)),
        ],
        out_specs=pl.BlockSpec((BE, C), lambda i: (i, 0)),
        out_shape=jax.ShapeDtypeStruct((EP_CK, C), jnp.float32),
    )(g, we1r, be1r, w2t, be2r, wc1t, bc1r, wc2t)


def _sc_scatter_body(et_hbm, rowp_hbm, z_hbm, acc_out,
                     idxr_v, et0, et1, acc, sem0, sem1):
    cid = lax.axis_index("c")
    sid = lax.axis_index("s")
    wid = sid * NC + cid
    wbase = wid * WINS_CK
    r0 = sid * NROWS_TILE

    pltpu.sync_copy(z_hbm, acc.at[pl.ds(r0, NROWS_TILE)])
    pltpu.sync_copy(rowp_hbm.at[pl.ds(wbase, WINS_CK)], idxr_v)
    plsc.subcore_barrier()

    et_v = (et0, et1)
    sem = (sem0, sem1)

    def fire(j, b):
        ebase = wid * EPW_CK + j * W
        pltpu.async_copy(et_hbm.at[pl.ds(ebase, W)], et_v[b], sem[b])

    def drain(b):
        pltpu.make_async_copy(et_hbm.at[pl.ds(0, W)], et_v[b],
                              sem[b]).wait()

    fire(0, 0)

    def superwin(js, carry):
        for bb in range(2):
            j = js * 2 + bb
            jn = jnp.minimum(j + 1, WINS_CK - 1)
            fire(jn, (bb + 1) % 2)
            drain(bb)
            pltpu.sync_copy(et_v[bb], acc.at[idxr_v.at[j]], add=True)
        return carry

    lax.fori_loop(0, WINS_CK // 2, superwin, 0)
    drain(0)
    plsc.subcore_barrier()

    pltpu.sync_copy(acc.at[pl.ds(r0, NROWS_TILE)],
                    acc_out.at[cid, pl.ds(r0, NROWS_TILE)])


def _sc_scatter(et, rowp):
    mesh = plsc.VectorSubcoreMesh(core_axis_name="c", subcore_axis_name="s")
    fn = pl.kernel(
        _sc_scatter_body,
        out_type=jax.ShapeDtypeStruct((NC, NP, C), jnp.float32),
        mesh=mesh,
        scratch_types=[
            pltpu.VMEM((WINS_CK, W), jnp.int32),
            pltpu.VMEM((W, C), jnp.float32),
            pltpu.VMEM((W, C), jnp.float32),
            pltpu.VMEM_SHARED((NP, C), jnp.float32),
            pltpu.SemaphoreType.DMA,
            pltpu.SemaphoreType.DMA,
        ],
        compiler_params=pltpu.CompilerParams(needs_layout_passes=False),
    )
    z = jnp.zeros((NROWS_TILE, C), jnp.float32)
    return fn(et, rowp, z)


def _node_body(h_ref, x_ref, accp_ref, wn1at_ref, wn1bt_ref,
               bn1_ref, wn2t_ref, bn2_ref, ho_ref, xo_ref):
    accp = accp_ref[...]
    acc = accp[0]
    for k in range(1, NCHUNK * NC):
        acc = acc + accp[k]
    msg = acc[:, 0:H]
    t = jnp.dot(h_ref[...], wn1at_ref[...],
                preferred_element_type=jnp.float32) \
        + jnp.dot(msg, wn1bt_ref[...], preferred_element_type=jnp.float32) \
        + bn1_ref[...]
    t = t * jax.nn.sigmoid(t)
    ho_ref[...] = jnp.dot(t, wn2t_ref[...],
                          preferred_element_type=jnp.float32) + bn2_ref[...]
    xo_ref[...] = x_ref[...] + acc[:, H:H + 3]


def _node_mlp(h, x, accp, wn1at, wn1bt, bn1r, wn2t, bn2r):
    return pl.pallas_call(
        _node_body,
        grid=(N // BN,),
        in_specs=[
            pl.BlockSpec((BN, D), lambda i: (i, 0)),
            pl.BlockSpec((BN, 3), lambda i: (i, 0)),
            pl.BlockSpec((NCHUNK * NC, BN, C), lambda i: (0, i, 0)),
            _full_spec((D, H)),
            _full_spec((H, H)),
            _full_spec((1, H)),
            _full_spec((H, D)),
            _full_spec((1, D)),
        ],
        out_specs=[
            pl.BlockSpec((BN, D), lambda i: (i, 0)),
            pl.BlockSpec((BN, 3), lambda i: (i, 0)),
        ],
        out_shape=[
            jax.ShapeDtypeStruct((N, D), jnp.float32),
            jax.ShapeDtypeStruct((N, 3), jnp.float32),
        ],
    )(h, x, accp, wn1at, wn1bt, bn1r, wn2t, bn2r)


def kernel(h, x, edge_index, box_edges, We1, be1, We2, be2,
           Wn1, bn1, Wn2, bn2, Wc1, bc1, Wc2):
    f32 = jnp.float32
    row = edge_index[0].astype(jnp.int32)
    col = edge_index[1].astype(jnp.int32)
    pad_t = N + (jnp.arange(EP - E, dtype=jnp.int32) % PAD_ROWS)
    rowp = jnp.concatenate([row, pad_t]).reshape(EP // W, W)
    colp = jnp.concatenate([col, pad_t]).reshape(EP // W, W)
    xtp = jnp.pad(x.astype(f32).T, ((0, 0), (0, PAD_ROWS)))

    wab = jnp.concatenate([We1[:, :D].T, We1[:, D:2 * D].T], axis=1)
    we1r = We1[:, 2 * D].reshape(1, H)
    be1r = be1.reshape(1, H)
    w2t = We2.T.astype(jnp.bfloat16)
    be2r = be2.reshape(1, H)
    wc1t = Wc1.T.astype(jnp.bfloat16)
    bc1r = bc1.reshape(1, H)
    wc2t = Wc2.T
    boxf = box_edges.astype(f32)
    pbc = jnp.concatenate([jnp.repeat(boxf, 16),
                           jnp.repeat(1.0 / boxf, 16)])
    wn1at = Wn1[:, :D].T
    wn1bt = Wn1[:, D:].T
    bn1r = bn1.reshape(1, H)
    wn2t = Wn2.T
    bn2r = bn2.reshape(1, D)

    ab = _preproj(h, wab)
    abp = jnp.pad(ab, ((0, PAD_ROWS), (0, 0)))
    rck = EP_CK // W
    accs = []
    for c in range(NCHUNK):
        rowc = rowp[c * rck:(c + 1) * rck]
        colc = colp[c * rck:(c + 1) * rck]
        g = _sc_gather(abp, xtp, rowc, colc, pbc)
        et = _edge_mlp(g, we1r, be1r, w2t, be2r, wc1t, bc1r, wc2t)
        accs.append(_sc_scatter(et, rowc))
    accp = jnp.concatenate(accs, axis=0)
    h_out, x_out = _node_mlp(h, x, accp, wn1at, wn1bt, bn1r, wn2t, bn2r)
    return (h_out, x_out)


# speedup vs baseline: 1.1093x; 1.1093x over previous
import jax
import jax.numpy as jnp
from jax import lax
from jax.experimental import pallas as pl
from jax.experimental.pallas import tpu as pltpu
from jax.experimental.pallas import tpu_sc as plsc

N = 10000
E = 320000
D = 128
H = 64
C = 128
NC = 2
NS = 16
NW = NC * NS
PAD_ROWS = 240
NP = N + PAD_ROWS
W = 128
WINS = 80
EPW = W * WINS
EP = NW * EPW
NROWS_TILE = NP // NS
BE = 2048
BN = 1000
NCHUNK = 2
WINS_CK = WINS // NCHUNK
EPW_CK = W * WINS_CK
EP_CK = EP // NCHUNK


def _full_spec(shape):
    return pl.BlockSpec(shape, lambda i: tuple(0 for _ in shape))


def _preproj_body(h_ref, w_ref, ab_ref):
    ab_ref[...] = jnp.dot(h_ref[...], w_ref[...],
                          preferred_element_type=jnp.float32)


def _preproj(h, w):
    return pl.pallas_call(
        _preproj_body,
        grid=(N // BN,),
        in_specs=[
            pl.BlockSpec((BN, D), lambda i: (i, 0)),
            _full_spec((D, C)),
        ],
        out_specs=pl.BlockSpec((BN, C), lambda i: (i, 0)),
        out_shape=jax.ShapeDtypeStruct((N, C), jnp.float32),
    )(h, w)


def _sc_gather_body(ab_hbm, xt0_hbm, xt1_hbm, xt2_hbm, rowp_hbm, colp_hbm,
                    pbc_hbm, g_hbm,
                    idxr_v, idxc_v, xp0, xp1, xp2, pbc_v,
                    abrow0, abrow1, abcol0, abcol1,
                    sema0, sema1, semb0, semb1):
    cid = lax.axis_index("c")
    sid = lax.axis_index("s")
    wid = sid * NC + cid
    wbase = wid * WINS_CK
    pltpu.sync_copy(rowp_hbm.at[pl.ds(wbase, WINS_CK)], idxr_v)
    pltpu.sync_copy(colp_hbm.at[pl.ds(wbase, WINS_CK)], idxc_v)
    pltpu.sync_copy(xt0_hbm, xp0)
    pltpu.sync_copy(xt1_hbm, xp1)
    pltpu.sync_copy(xt2_hbm, xp2)
    pltpu.sync_copy(pbc_hbm, pbc_v)
    box = tuple(pbc_v[pl.ds(p * 16, 16)] for p in range(3))
    ibox = tuple(pbc_v[pl.ds(48 + p * 16, 16)] for p in range(3))
    magic = jnp.full((16,), 12582912.0, jnp.float32)

    abrow = (abrow0, abrow1)
    abcol = (abcol0, abcol1)
    sema = (sema0, sema1)
    semb = (semb0, semb1)

    def fire(j, b):
        pltpu.async_copy(ab_hbm.at[idxr_v.at[j]], abrow[b], sema[b])
        pltpu.async_copy(ab_hbm.at[idxc_v.at[j]], abcol[b], semb[b])

    def drain(b):
        pltpu.make_async_copy(ab_hbm.at[idxr_v.at[0]], abrow[b],
                              sema[b]).wait()
        pltpu.make_async_copy(ab_hbm.at[idxc_v.at[0]], abcol[b],
                              semb[b]).wait()

    fire(0, 0)

    def process(j, b):
        drain(b)

        def addrows(r4, c2):
            for dr in range(4):
                r = r4 * 4 + dr
                for q in range(H // 16):
                    plsc.addupdate(abrow[b].at[r, pl.ds(q * 16, 16)],
                                   abcol[b][r, pl.ds(H + q * 16, 16)])
            return c2

        lax.fori_loop(0, W // 4, addrows, 0)

        for g in range(W // 16):
            s = pl.ds(g * 16, 16)
            ev = jnp.arange(g * 16, g * 16 + 16, dtype=jnp.int32)
            rv = idxr_v[j, s]
            cv = idxc_v[j, s]
            d = []
            for p, plane in enumerate((xp0, xp1, xp2)):
                dp = plsc.load_gather(plane, [rv]) \
                    - plsc.load_gather(plane, [cv])
                k = (dp * ibox[p] + magic) - magic
                dp = dp - box[p] * k
                d.append(dp)
                pv = jnp.full((16,), H + p, dtype=jnp.int32)
                plsc.store_scatter(abrow[b], [ev, pv], dp)
            r2 = d[0] * d[0] + d[1] * d[1] + d[2] * d[2]
            pv = jnp.full((16,), H + 3, dtype=jnp.int32)
            plsc.store_scatter(abrow[b], [ev, pv], r2)
        ebase = wid * EPW_CK + j * W
        pltpu.sync_copy(abrow[b], g_hbm.at[pl.ds(ebase, W)])

    def superwin(js, carry):
        for bb in range(2):
            j = js * 2 + bb
            jn = jnp.minimum(j + 1, WINS_CK - 1)
            fire(jn, (bb + 1) % 2)
            process(j, bb)
        return carry

    lax.fori_loop(0, WINS_CK // 2, superwin, 0)
    drain(0)


def _sc_gather(abp, xtp, rowp, colp, pbc):
    mesh = plsc.VectorSubcoreMesh(core_axis_name="c", subcore_axis_name="s")
    fn = pl.kernel(
        _sc_gather_body,
        out_type=jax.ShapeDtypeStruct((EP_CK, C), jnp.float32),
        mesh=mesh,
        scratch_types=[
            pltpu.VMEM((WINS_CK, W), jnp.int32),
            pltpu.VMEM((WINS_CK, W), jnp.int32),
            pltpu.VMEM((NP,), jnp.float32),
            pltpu.VMEM((NP,), jnp.float32),
            pltpu.VMEM((NP,), jnp.float32),
            pltpu.VMEM((96,), jnp.float32),
            pltpu.VMEM((W, C), jnp.float32),
            pltpu.VMEM((W, C), jnp.float32),
            pltpu.VMEM((W, C), jnp.float32),
            pltpu.VMEM((W, C), jnp.float32),
            pltpu.SemaphoreType.DMA,
            pltpu.SemaphoreType.DMA,
            pltpu.SemaphoreType.DMA,
            pltpu.SemaphoreType.DMA,
        ],
        compiler_params=pltpu.CompilerParams(needs_layout_passes=False),
    )
    return fn(abp, xtp[0], xtp[1], xtp[2], rowp, colp, pbc)


def _edge_body(g_ref, we1r_ref, be1_ref, w2t_ref, be2_ref,
               wc1t_ref, bc1_ref, wc2t_ref, et_ref):
    bf16 = jnp.bfloat16

    def silu(v):
        return v * (0.5 * jnp.tanh(0.5 * v) + 0.5)

    gblk = g_ref[...]
    d = gblk[:, H:H + 3]
    r2c = gblk[:, H + 3:H + 4]
    r2f = jnp.dot(r2c, jnp.ones((1, H), jnp.float32),
                  preferred_element_type=jnp.float32)
    normf = jnp.sqrt(r2f + 1e-8)
    pre = gblk[:, 0:H] + normf * we1r_ref[...] + be1_ref[...]
    t1 = silu(pre)
    t2 = jnp.dot(t1.astype(bf16), w2t_ref[...],
                 preferred_element_type=jnp.float32) + be2_ref[...]
    ef = silu(t2)
    s1 = jnp.dot(ef.astype(bf16), wc1t_ref[...],
                 preferred_element_type=jnp.float32) + bc1_ref[...]
    s1 = silu(s1)
    scale = jnp.dot(s1, wc2t_ref[...], preferred_element_type=jnp.float32)
    m = scale / (normf[:, 0:1] + 1.0)
    tr = d * m
    et_ref[...] = jnp.concatenate(
        [ef, tr, jnp.zeros((ef.shape[0], C - H - 3), jnp.float32)], axis=1)


def _edge_mlp(g, we1r, be1r, w2t, be2r, wc1t, bc1r, wc2t):
    return pl.pallas_call(
        _edge_body,
        grid=(EP_CK // BE,),
        in_specs=[
            pl.BlockSpec((BE, C), lambda i: (i, 0)),
            _full_spec((1, H)),
            _full_spec((1, H)),
            _full_spec((H, H)),
            _full_spec((1, H)),
            _full_spec((H, H)),
            _full_spec((1, H)),
            _full_spec((H, 1)),
        ],
        out_specs=pl.BlockSpec((BE, C), lambda i: (i, 0)),
        out_shape=jax.ShapeDtypeStruct((EP_CK, C), jnp.float32),
    )(g, we1r, be1r, w2t, be2r, wc1t, bc1r, wc2t)


def _sc_scatter_body(et_hbm, rowp_hbm, z_hbm, acc_out,
                     idxr_v, et0, et1, acc, sem0, sem1):
    cid = lax.axis_index("c")
    sid = lax.axis_index("s")
    wid = sid * NC + cid
    wbase = wid * WINS_CK
    r0 = sid * NROWS_TILE

    pltpu.sync_copy(z_hbm, acc.at[pl.ds(r0, NROWS_TILE)])
    pltpu.sync_copy(rowp_hbm.at[pl.ds(wbase, WINS_CK)], idxr_v)
    plsc.subcore_barrier()

    et_v = (et0, et1)
    sem = (sem0, sem1)

    def fire(j, b):
        ebase = wid * EPW_CK + j * W
        pltpu.async_copy(et_hbm.at[pl.ds(ebase, W)], et_v[b], sem[b])

    def drain(b):
        pltpu.make_async_copy(et_hbm.at[pl.ds(0, W)], et_v[b],
                              sem[b]).wait()

    fire(0, 0)

    def superwin(js, carry):
        for bb in range(2):
            j = js * 2 + bb
            jn = jnp.minimum(j + 1, WINS_CK - 1)
            fire(jn, (bb + 1) % 2)
            drain(bb)
            pltpu.sync_copy(et_v[bb], acc.at[idxr_v.at[j]], add=True)
        return carry

    lax.fori_loop(0, WINS_CK // 2, superwin, 0)
    drain(0)
    plsc.subcore_barrier()

    pltpu.sync_copy(acc.at[pl.ds(r0, NROWS_TILE)],
                    acc_out.at[cid, pl.ds(r0, NROWS_TILE)])


def _sc_scatter(et, rowp):
    mesh = plsc.VectorSubcoreMesh(core_axis_name="c", subcore_axis_name="s")
    fn = pl.kernel(
        _sc_scatter_body,
        out_type=jax.ShapeDtypeStruct((NC, NP, C), jnp.float32),
        mesh=mesh,
        scratch_types=[
            pltpu.VMEM((WINS_CK, W), jnp.int32),
            pltpu.VMEM((W, C), jnp.float32),
            pltpu.VMEM((W, C), jnp.float32),
            pltpu.VMEM_SHARED((NP, C), jnp.float32),
            pltpu.SemaphoreType.DMA,
            pltpu.SemaphoreType.DMA,
        ],
        compiler_params=pltpu.CompilerParams(needs_layout_passes=False),
    )
    z = jnp.zeros((NROWS_TILE, C), jnp.float32)
    return fn(et, rowp, z)


def _node_body(h_ref, x_ref, accp_ref, wn1at_ref, wn1bt_ref,
               bn1_ref, wn2t_ref, bn2_ref, ho_ref, xo_ref):
    accp = accp_ref[...]
    acc = accp[0]
    for k in range(1, NCHUNK * NC):
        acc = acc + accp[k]
    msg = acc[:, 0:H]
    t = jnp.dot(h_ref[...], wn1at_ref[...],
                preferred_element_type=jnp.float32) \
        + jnp.dot(msg, wn1bt_ref[...], preferred_element_type=jnp.float32) \
        + bn1_ref[...]
    t = t * jax.nn.sigmoid(t)
    ho_ref[...] = jnp.dot(t, wn2t_ref[...],
                          preferred_element_type=jnp.float32) + bn2_ref[...]
    xo_ref[...] = x_ref[...] + acc[:, H:H + 3]


def _node_mlp(h, x, accp, wn1at, wn1bt, bn1r, wn2t, bn2r):
    return pl.pallas_call(
        _node_body,
        grid=(N // BN,),
        in_specs=[
            pl.BlockSpec((BN, D), lambda i: (i, 0)),
            pl.BlockSpec((BN, 3), lambda i: (i, 0)),
            pl.BlockSpec((NCHUNK * NC, BN, C), lambda i: (0, i, 0)),
            _full_spec((D, H)),
            _full_spec((H, H)),
            _full_spec((1, H)),
            _full_spec((H, D)),
            _full_spec((1, D)),
        ],
        out_specs=[
            pl.BlockSpec((BN, D), lambda i: (i, 0)),
            pl.BlockSpec((BN, 3), lambda i: (i, 0)),
        ],
        out_shape=[
            jax.ShapeDtypeStruct((N, D), jnp.float32),
            jax.ShapeDtypeStruct((N, 3), jnp.float32),
        ],
    )(h, x, accp, wn1at, wn1bt, bn1r, wn2t, bn2r)


def kernel(h, x, edge_index, box_edges, We1, be1, We2, be2,
           Wn1, bn1, Wn2, bn2, Wc1, bc1, Wc2):
    f32 = jnp.float32
    row = edge_index[0].astype(jnp.int32)
    col = edge_index[1].astype(jnp.int32)
    pad_t = N + (jnp.arange(EP - E, dtype=jnp.int32) % PAD_ROWS)
    rowp = jnp.concatenate([row, pad_t]).reshape(EP // W, W)
    colp = jnp.concatenate([col, pad_t]).reshape(EP // W, W)
    xtp = jnp.pad(x.astype(f32).T, ((0, 0), (0, PAD_ROWS)))

    wab = jnp.concatenate([We1[:, :D].T, We1[:, D:2 * D].T], axis=1)
    we1r = We1[:, 2 * D].reshape(1, H)
    be1r = be1.reshape(1, H)
    w2t = We2.T.astype(jnp.bfloat16)
    be2r = be2.reshape(1, H)
    wc1t = Wc1.T.astype(jnp.bfloat16)
    bc1r = bc1.reshape(1, H)
    wc2t = Wc2.T
    boxf = box_edges.astype(f32)
    pbc = jnp.concatenate([jnp.repeat(boxf, 16),
                           jnp.repeat(1.0 / boxf, 16)])
    wn1at = Wn1[:, :D].T
    wn1bt = Wn1[:, D:].T
    bn1r = bn1.reshape(1, H)
    wn2t = Wn2.T
    bn2r = bn2.reshape(1, D)

    ab = _preproj(h, wab)
    abp = jnp.pad(ab, ((0, PAD_ROWS), (0, 0)))
    rck = EP_CK // W
    accs = []
    for c in range(NCHUNK):
        rowc = rowp[c * rck:(c + 1) * rck]
        colc = colp[c * rck:(c + 1) * rck]
        g = _sc_gather(abp, xtp, rowc, colc, pbc)
        et = _edge_mlp(g, we1r, be1r, w2t, be2r, wc1t, bc1r, wc2t)
        accs.append(_sc_scatter(et, rowc))
    accp = jnp.concatenate(accs, axis=0)
    h_out, x_out = _node_mlp(h, x, accp, wn1at, wn1bt, bn1r, wn2t, bn2r)
    return (h_out, x_out)
